# baseline (speedup 1.0000x reference)
import functools

import jax
import jax.numpy as jnp
from jax import lax
from jax.experimental import pallas as pl
from jax.experimental.pallas import tpu as pltpu
from jax.experimental.pallas import tpu_sc as plsc

_NUM_CORES = 2
_NUM_SUBCORES = 16
_NUM_WORKERS = _NUM_CORES * _NUM_SUBCORES

_CHUNK = 128

_BC = 2048


def _transpose_block(t_ref, o_ref):
    o_ref[...] = jnp.swapaxes(t_ref[...], 0, 1)


def _relayout(table_t):
    dim, num_classes = table_t.shape
    grid = (num_classes + _BC - 1) // _BC
    return pl.pallas_call(
        _transpose_block,
        grid=(grid,),
        in_specs=[pl.BlockSpec((dim, _BC), lambda g: (0, g))],
        out_specs=pl.BlockSpec((_BC, dim), lambda g: (g, 0)),
        out_shape=jax.ShapeDtypeStruct((num_classes, dim), jnp.float32),
    )(table_t)


def _make_gather(batch: int, dim: int):
    b_per_w = batch // _NUM_WORKERS
    n_chunks = b_per_w // _CHUNK
    mesh = plsc.VectorSubcoreMesh(core_axis_name="c", subcore_axis_name="s")

    @functools.partial(
        pl.kernel,
        out_type=jax.ShapeDtypeStruct((batch, dim), jnp.float32),
        mesh=mesh,
        scratch_types=[
            pltpu.VMEM((b_per_w,), jnp.int32),
            pltpu.VMEM((b_per_w, dim), jnp.float32),
            pltpu.SemaphoreType.DMA,
        ],
        compiler_params=pltpu.CompilerParams(use_tc_tiling_on_sc=False),
    )
    def gather_kernel(table_hbm, idx_hbm, out_hbm, idx_v, rows_v, sem):
        wid = lax.axis_index("s") * _NUM_CORES + lax.axis_index("c")
        base = wid * b_per_w
        pltpu.sync_copy(idx_hbm.at[pl.ds(base, b_per_w)], idx_v)
        copies = [
            pltpu.async_copy(
                table_hbm.at[idx_v.at[pl.ds(j * _CHUNK, _CHUNK)]],
                rows_v.at[pl.ds(j * _CHUNK, _CHUNK)],
                sem,
            )
            for j in range(n_chunks)
        ]
        for c in copies:
            c.wait()
        pltpu.sync_copy(rows_v, out_hbm.at[pl.ds(base, b_per_w)])

    return gather_kernel


def kernel(categorical, table):
    (batch,) = categorical.shape
    _, dim = table.shape
    table_rm = _relayout(jnp.transpose(table))
    return _make_gather(batch, dim)(table_rm, categorical.astype(jnp.int32))


# speedup vs baseline: 4.5292x; 4.5292x over previous
import functools

import jax
import jax.numpy as jnp
from jax import lax
from jax.experimental import pallas as pl
from jax.experimental.pallas import tpu as pltpu
from jax.experimental.pallas import tpu_sc as plsc

_NCORE = 2
_NSUB = 16
_NW = _NCORE * _NSUB
_L = 16

_SLAB = 512
_CHM = 32
_POS_BITS = 14
_HUGE = 2**30


def _iota16():
    return lax.iota(jnp.int32, _L)


def _popcnt(mask):
    return plsc.all_reduce_population_count(mask)


def _make_sweep(batch: int, num_classes: int, dim: int):
    tcols = num_classes // 128
    tail = num_classes - tcols * 128
    nslab = tcols * 128 // _SLAB
    per = nslab // _NW
    rem = nslab - per * _NW
    mesh = plsc.VectorSubcoreMesh(core_axis_name="c", subcore_axis_name="s")

    @functools.partial(
        pl.kernel,
        out_type=jax.ShapeDtypeStruct((batch, 2 * dim), jnp.float32),
        mesh=mesh,
        scratch_types=[
            pltpu.VMEM((batch,), jnp.int32),
            pltpu.VMEM((batch + _L,), jnp.int32),
            pltpu.VMEM((2, dim, _SLAB), jnp.float32),
            pltpu.VMEM((dim, 128), jnp.float32),
            pltpu.VMEM((_CHM,), jnp.int32),
            pltpu.VMEM((2, _CHM, 2 * dim), jnp.float32),
            pltpu.VMEM((2, 1, _CHM), jnp.int32),
            pltpu.SemaphoreType.DMA,
            pltpu.SemaphoreType.DMA,
            pltpu.SemaphoreType.DMA,
            pltpu.SemaphoreType.DMA,
        ],
        compiler_params=pltpu.CompilerParams(needs_layout_passes=False),
    )
    def sweep_kernel(
        table_t,
        idx_hbm,
        tail_hbm,
        out_hbm,
        idx_all,
        hits,
        slabs,
        tailslab,
        chunkbuf,
        staged,
        posbuf,
        sem_s0,
        sem_s1,
        sem_o0,
        sem_o1,
    ):
        wid = lax.axis_index("s") * _NCORE + lax.axis_index("c")
        last = wid == _NW - 1
        sbase = wid * per
        nsl = lax.select(last, per + rem, per)
        lo = sbase * _SLAB
        hi = lax.select(last, jnp.int32(num_classes), lo + nsl * _SLAB)

        pltpu.sync_copy(idx_hbm, idx_all)
        iota = _iota16()

        def filt(g, cnt):
            v = idx_all[pl.ds(g * _L, _L)]
            m = (v >= lo) & (v < hi)
            packed = ((v - lo) << _POS_BITS) | (jnp.full((_L,), g * _L, jnp.int32) + iota)
            r = plsc.cumsum(m.astype(jnp.int32)) + cnt
            plsc.store_scatter(hits, [r - 1], packed, mask=m)
            return cnt + _popcnt(m)

        cnt = lax.fori_loop(0, batch // _L, filt, jnp.zeros((_L,), jnp.int32))
        ht = cnt[0]
        gg = (ht // _L) * _L
        gg = pl.multiple_of(gg, _L)
        vg = hits[pl.ds(gg, _L)]
        hits[pl.ds(gg, _L)] = jnp.where(
            iota < ht - gg, vg, jnp.full((_L,), _HUGE, jnp.int32)
        )
        ngrp = (ht + _L - 1) // _L

        def slab_count(wlo, whi):
            def body(g, c):
                v = hits[pl.ds(g * _L, _L)]
                m = (v >= wlo) & (v < whi)
                return c + _popcnt(m)

            return lax.fori_loop(0, ngrp, body, jnp.zeros((_L,), jnp.int32))[0]

        def extract_chunk(wlo, whi, c):
            rlo = c * _CHM

            def body(g, r):
                v = hits[pl.ds(g * _L, _L)]
                m = (v >= wlo) & (v < whi)
                rk = plsc.cumsum(m.astype(jnp.int32)) + r
                sel = m & (rk > rlo) & (rk <= rlo + _CHM)
                plsc.store_scatter(chunkbuf, [rk - 1 - rlo], v, mask=sel)
                return r + _popcnt(m)

            lax.fori_loop(0, ngrp, body, jnp.zeros((_L,), jnp.int32))

        def drain_out(q_is0, fired):
            @pl.when(q_is0 & (fired == 1))
            def _():
                pltpu.make_async_copy(
                    out_hbm.at[pl.ds(0, _CHM)], staged.at[0], sem_o0
                ).wait()

        def drain_out1(q_is1, fired):
            @pl.when(q_is1 & (fired == 1))
            def _():
                pltpu.make_async_copy(
                    out_hbm.at[pl.ds(0, _CHM)], staged.at[1], sem_o1
                ).wait()

        def gather_chunk(colbase, hc, q, from_tail, p):
            lastv = plsc.load_gather(chunkbuf, [jnp.full((_L,), hc - 1, jnp.int32)])
            qv = jnp.full((_L,), q, jnp.int32)
            for g2 in range(_CHM // _L):
                lanes = jnp.full((_L,), g2 * _L, jnp.int32) + iota
                pv = chunkbuf[pl.ds(g2 * _L, _L)]
                pvf = jnp.where(lanes < hc, pv, lastv)
                col = (pvf >> _POS_BITS) - colbase
                pos = pvf & jnp.int32((1 << _POS_BITS) - 1)
                plsc.store_scatter(posbuf, [qv, jnp.zeros((_L,), jnp.int32), lanes], pos)
                pvec = jnp.full((_L,), p, jnp.int32)
                for j in range(dim):
                    jv = jnp.full((_L,), j, jnp.int32)

                    @pl.when(~from_tail)
                    def _():
                        vals = plsc.load_gather(slabs, [pvec, jv, col])
                        plsc.store_scatter(staged, [qv, lanes, jv], vals)

                    @pl.when(from_tail)
                    def _():
                        vals = plsc.load_gather(tailslab, [jv, col])
                        plsc.store_scatter(staged, [qv, lanes, jv], vals)

        def fire_out(q):
            @pl.when(q == 0)
            def _():
                pltpu.async_copy(staged.at[0], out_hbm.at[posbuf.at[0, 0]], sem_o0)

            @pl.when(q == 1)
            def _():
                pltpu.async_copy(staged.at[1], out_hbm.at[posbuf.at[1, 0]], sem_o1)

        def slab_src(s):
            off = pl.multiple_of((sbase + s) * _SLAB, _SLAB)
            return table_t.at[:, pl.ds(off, _SLAB)]

        @pl.when(nsl > 0)
        def _():
            pltpu.async_copy(slab_src(0), slabs.at[0], sem_s0)

        def slab_body(i, carry):
            cpar, f0, f1 = carry
            nxt = i + 1

            @pl.when((nxt < nsl) & (nxt % 2 == 0))
            def _():
                pltpu.async_copy(slab_src(nxt), slabs.at[0], sem_s0)

            @pl.when((nxt < nsl) & (nxt % 2 == 1))
            def _():
                pltpu.async_copy(slab_src(nxt), slabs.at[1], sem_s1)

            @pl.when(i % 2 == 0)
            def _():
                pltpu.make_async_copy(slab_src(i), slabs.at[0], sem_s0).wait()

            @pl.when(i % 2 == 1)
            def _():
                pltpu.make_async_copy(slab_src(i), slabs.at[1], sem_s1).wait()

            wlo = i << 23
            whi = nxt << 23
            hs = slab_count(wlo, whi)
            nch = (hs + _CHM - 1) // _CHM

            def chunk_body(c, carry2):
                cpar2, g0, g1 = carry2
                q = cpar2 % 2
                extract_chunk(wlo, whi, c)
                hc = jnp.minimum(hs - c * _CHM, _CHM)
                drain_out(q == 0, g0)
                drain_out1(q == 1, g1)
                gather_chunk(i * _SLAB, hc, q, jnp.bool_(False), i % 2)
                fire_out(q)
                g0n = lax.select(q == 0, jnp.int32(1), g0)
                g1n = lax.select(q == 1, jnp.int32(1), g1)
                return (cpar2 + 1, g0n, g1n)

            return lax.fori_loop(0, nch, chunk_body, (cpar, f0, f1))

        cpar, f0, f1 = lax.fori_loop(
            0, nsl, slab_body, (jnp.int32(0), jnp.int32(0), jnp.int32(0))
        )

        drain_out(jnp.bool_(True), f0)
        drain_out1(jnp.bool_(True), f1)

        if tail > 0:

            @pl.when(last)
            def _():
                pltpu.sync_copy(tail_hbm, tailslab)
                wlo = (per + rem) << 23
                hs = slab_count(wlo, _HUGE)
                nch = (hs + _CHM - 1) // _CHM

                def tail_chunk(c, _):
                    extract_chunk(wlo, _HUGE, c)
                    hc = jnp.minimum(hs - c * _CHM, _CHM)
                    gather_chunk((per + rem) * _SLAB, hc, jnp.int32(0), jnp.bool_(True), 0)
                    pltpu.async_copy(
                        staged.at[0], out_hbm.at[posbuf.at[0, 0]], sem_o0
                    ).wait()
                    return 0

                lax.fori_loop(0, nch, tail_chunk, 0)

    return sweep_kernel


def kernel(categorical, table):
    (batch,) = categorical.shape
    num_classes, dim = table.shape
    fn = _make_sweep(batch, num_classes, dim)
    tcols = num_classes // 128
    tail = num_classes - tcols * 128
    tail_rows = jnp.pad(table[tcols * 128 :, :], ((0, 128 - tail), (0, 0)))
    wide = fn(
        jnp.transpose(table),
        categorical.astype(jnp.int32),
        jnp.transpose(tail_rows),
    )
    return wide[:, :dim]


# speedup vs baseline: 4.5336x; 1.0010x over previous
import functools

import jax
import jax.numpy as jnp
from jax import lax
from jax.experimental import pallas as pl
from jax.experimental.pallas import tpu as pltpu
from jax.experimental.pallas import tpu_sc as plsc

_NCORE = 2
_NSUB = 16
_NW = _NCORE * _NSUB
_L = 16

_SLAB = 512
_CHM = 32
_POS_BITS = 14
_HUGE = 2**30


def _iota16():
    return lax.iota(jnp.int32, _L)


def _popcnt(mask):
    return plsc.all_reduce_population_count(mask)


def _make_sweep(batch: int, num_classes: int, dim: int):
    tcols = num_classes // 128
    tail = num_classes - tcols * 128
    nslab = tcols * 128 // _SLAB
    per = nslab // _NW
    rem = nslab - per * _NW
    mesh = plsc.VectorSubcoreMesh(core_axis_name="c", subcore_axis_name="s")

    @functools.partial(
        pl.kernel,
        out_type=jax.ShapeDtypeStruct((batch, 2 * dim), jnp.float32),
        mesh=mesh,
        scratch_types=[
            pltpu.VMEM((batch,), jnp.int32),
            pltpu.VMEM((batch + _L,), jnp.int32),
            pltpu.VMEM((2, dim, _SLAB), jnp.float32),
            pltpu.VMEM((dim, 128), jnp.float32),
            pltpu.VMEM((_CHM,), jnp.int32),
            pltpu.VMEM((2, _CHM, 2 * dim), jnp.float32),
            pltpu.VMEM((2, 1, _CHM), jnp.int32),
            pltpu.SemaphoreType.DMA,
            pltpu.SemaphoreType.DMA,
            pltpu.SemaphoreType.DMA,
            pltpu.SemaphoreType.DMA,
        ],
        compiler_params=pltpu.CompilerParams(needs_layout_passes=False),
    )
    def sweep_kernel(
        table_t,
        idx_hbm,
        tail_hbm,
        out_hbm,
        idx_all,
        hits,
        slabs,
        tailslab,
        chunkbuf,
        staged,
        posbuf,
        sem_s0,
        sem_s1,
        sem_o0,
        sem_o1,
    ):
        wid = lax.axis_index("s") * _NCORE + lax.axis_index("c")
        last = wid == _NW - 1
        sbase = wid * per
        nsl = lax.select(last, per + rem, per)
        lo = sbase * _SLAB
        hi = lax.select(last, jnp.int32(num_classes), lo + nsl * _SLAB)

        pltpu.sync_copy(idx_hbm, idx_all)
        iota = _iota16()

        def filt(g, cnt):
            v = idx_all[pl.ds(g * _L, _L)]
            m = (v >= lo) & (v < hi)
            packed = ((v - lo) << _POS_BITS) | (jnp.full((_L,), g * _L, jnp.int32) + iota)
            r = plsc.cumsum(m.astype(jnp.int32)) + cnt
            plsc.store_scatter(hits, [r - 1], packed, mask=m)
            return cnt + _popcnt(m)

        cnt = lax.fori_loop(0, batch // _L, filt, jnp.zeros((_L,), jnp.int32))
        ht = cnt[0]
        gg = (ht // _L) * _L
        gg = pl.multiple_of(gg, _L)
        vg = hits[pl.ds(gg, _L)]
        hits[pl.ds(gg, _L)] = jnp.where(
            iota < ht - gg, vg, jnp.full((_L,), _HUGE, jnp.int32)
        )
        ngrp = (ht + _L - 1) // _L

        def slab_count(wlo, whi):
            def body(g, c):
                v = hits[pl.ds(g * _L, _L)]
                m = (v >= wlo) & (v < whi)
                return c + _popcnt(m)

            return lax.fori_loop(0, ngrp, body, jnp.zeros((_L,), jnp.int32))[0]

        def extract_chunk(wlo, whi, c):
            rlo = c * _CHM

            def body(g, r):
                v = hits[pl.ds(g * _L, _L)]
                m = (v >= wlo) & (v < whi)
                rk = plsc.cumsum(m.astype(jnp.int32)) + r
                sel = m & (rk > rlo) & (rk <= rlo + _CHM)
                plsc.store_scatter(chunkbuf, [rk - 1 - rlo], v, mask=sel)
                return r + _popcnt(m)

            lax.fori_loop(0, ngrp, body, jnp.zeros((_L,), jnp.int32))

        def drain_out(q_is0, fired):
            @pl.when(q_is0 & (fired == 1))
            def _():
                pltpu.make_async_copy(
                    out_hbm.at[pl.ds(0, _CHM)], staged.at[0], sem_o0
                ).wait()

        def drain_out1(q_is1, fired):
            @pl.when(q_is1 & (fired == 1))
            def _():
                pltpu.make_async_copy(
                    out_hbm.at[pl.ds(0, _CHM)], staged.at[1], sem_o1
                ).wait()

        def gather_chunk(colbase, hc, q, from_tail, p):
            lastv = plsc.load_gather(chunkbuf, [jnp.full((_L,), hc - 1, jnp.int32)])
            qv = jnp.full((_L,), q, jnp.int32)
            for g2 in range(_CHM // _L):
                lanes = jnp.full((_L,), g2 * _L, jnp.int32) + iota
                pv = chunkbuf[pl.ds(g2 * _L, _L)]
                pvf = jnp.where(lanes < hc, pv, lastv)
                col = (pvf >> _POS_BITS) - colbase
                pos = pvf & jnp.int32((1 << _POS_BITS) - 1)
                plsc.store_scatter(posbuf, [qv, jnp.zeros((_L,), jnp.int32), lanes], pos)
                pvec = jnp.full((_L,), p, jnp.int32)
                for j in range(dim):
                    jv = jnp.full((_L,), j, jnp.int32)
                    if from_tail:
                        vals = plsc.load_gather(tailslab, [jv, col])
                    else:
                        vals = plsc.load_gather(slabs, [pvec, jv, col])
                    plsc.store_scatter(staged, [qv, lanes, jv], vals)

        def fire_out(q):
            @pl.when(q == 0)
            def _():
                pltpu.async_copy(staged.at[0], out_hbm.at[posbuf.at[0, 0]], sem_o0)

            @pl.when(q == 1)
            def _():
                pltpu.async_copy(staged.at[1], out_hbm.at[posbuf.at[1, 0]], sem_o1)

        def slab_src(s):
            off = pl.multiple_of((sbase + s) * _SLAB, _SLAB)
            return table_t.at[:, pl.ds(off, _SLAB)]

        @pl.when(nsl > 0)
        def _():
            pltpu.async_copy(slab_src(0), slabs.at[0], sem_s0)

        def slab_body(i, carry):
            cpar, f0, f1 = carry
            nxt = i + 1

            @pl.when((nxt < nsl) & (nxt % 2 == 0))
            def _():
                pltpu.async_copy(slab_src(nxt), slabs.at[0], sem_s0)

            @pl.when((nxt < nsl) & (nxt % 2 == 1))
            def _():
                pltpu.async_copy(slab_src(nxt), slabs.at[1], sem_s1)

            @pl.when(i % 2 == 0)
            def _():
                pltpu.make_async_copy(slab_src(i), slabs.at[0], sem_s0).wait()

            @pl.when(i % 2 == 1)
            def _():
                pltpu.make_async_copy(slab_src(i), slabs.at[1], sem_s1).wait()

            wlo = i << 23
            whi = nxt << 23
            hs = slab_count(wlo, whi)
            nch = (hs + _CHM - 1) // _CHM

            def chunk_body(c, carry2):
                cpar2, g0, g1 = carry2
                q = cpar2 % 2
                extract_chunk(wlo, whi, c)
                hc = jnp.minimum(hs - c * _CHM, _CHM)
                drain_out(q == 0, g0)
                drain_out1(q == 1, g1)
                gather_chunk(i * _SLAB, hc, q, False, i % 2)
                fire_out(q)
                g0n = lax.select(q == 0, jnp.int32(1), g0)
                g1n = lax.select(q == 1, jnp.int32(1), g1)
                return (cpar2 + 1, g0n, g1n)

            return lax.fori_loop(0, nch, chunk_body, (cpar, f0, f1))

        cpar, f0, f1 = lax.fori_loop(
            0, nsl, slab_body, (jnp.int32(0), jnp.int32(0), jnp.int32(0))
        )

        drain_out(jnp.bool_(True), f0)
        drain_out1(jnp.bool_(True), f1)

        if tail > 0:

            @pl.when(last)
            def _():
                pltpu.sync_copy(tail_hbm, tailslab)
                wlo = (per + rem) << 23
                hs = slab_count(wlo, _HUGE)
                nch = (hs + _CHM - 1) // _CHM

                def tail_chunk(c, _):
                    extract_chunk(wlo, _HUGE, c)
                    hc = jnp.minimum(hs - c * _CHM, _CHM)
                    gather_chunk((per + rem) * _SLAB, hc, jnp.int32(0), True, 0)
                    pltpu.async_copy(
                        staged.at[0], out_hbm.at[posbuf.at[0, 0]], sem_o0
                    ).wait()
                    return 0

                lax.fori_loop(0, nch, tail_chunk, 0)

    return sweep_kernel


def kernel(categorical, table):
    (batch,) = categorical.shape
    num_classes, dim = table.shape
    fn = _make_sweep(batch, num_classes, dim)
    tcols = num_classes // 128
    tail = num_classes - tcols * 128
    tail_rows = jnp.pad(table[tcols * 128 :, :], ((0, 128 - tail), (0, 0)))
    wide = fn(
        jnp.transpose(table),
        categorical.astype(jnp.int32),
        jnp.transpose(tail_rows),
    )
    return wide[:, :dim]


# speedup vs baseline: 4.7009x; 1.0369x over previous
import functools

import jax
import jax.numpy as jnp
from jax import lax
from jax.experimental import pallas as pl
from jax.experimental.pallas import tpu as pltpu
from jax.experimental.pallas import tpu_sc as plsc

_NCORE = 2
_NSUB = 16
_NW = _NCORE * _NSUB
_L = 16

_SLAB = 512
_CHM = 32
_POS_BITS = 14
_HUGE = 2**30


def _iota16():
    return lax.iota(jnp.int32, _L)


def _popcnt(mask):
    return plsc.all_reduce_population_count(mask)


def _make_sweep(batch: int, num_classes: int, dim: int):
    tcols = num_classes // 128
    tail = num_classes - tcols * 128
    nslab = tcols * 128 // _SLAB
    per = nslab // _NW
    rem = nslab - per * _NW
    mesh = plsc.VectorSubcoreMesh(core_axis_name="c", subcore_axis_name="s")

    @functools.partial(
        pl.kernel,
        out_type=jax.ShapeDtypeStruct((batch, 2 * dim), jnp.float32),
        mesh=mesh,
        scratch_types=[
            pltpu.VMEM((batch,), jnp.int32),
            pltpu.VMEM((batch + _L,), jnp.int32),
            pltpu.VMEM((2, dim, _SLAB), jnp.float32),
            pltpu.VMEM((dim, 128), jnp.float32),
            pltpu.VMEM((_CHM,), jnp.int32),
            pltpu.VMEM((2, _CHM, 2 * dim), jnp.float32),
            pltpu.VMEM((2, 1, _CHM), jnp.int32),
            pltpu.SemaphoreType.DMA,
            pltpu.SemaphoreType.DMA,
            pltpu.SemaphoreType.DMA,
            pltpu.SemaphoreType.DMA,
        ],
        compiler_params=pltpu.CompilerParams(needs_layout_passes=False),
    )
    def sweep_kernel(
        table_t,
        idx_hbm,
        tail_hbm,
        out_hbm,
        idx_all,
        hits,
        slabs,
        tailslab,
        chunkbuf,
        staged,
        posbuf,
        sem_s0,
        sem_s1,
        sem_o0,
        sem_o1,
    ):
        wid = lax.axis_index("s") * _NCORE + lax.axis_index("c")
        last = wid == _NW - 1
        sbase = wid * per
        nsl = lax.select(last, per + rem, per)
        lo = sbase * _SLAB
        hi = lax.select(last, jnp.int32(num_classes), lo + nsl * _SLAB)

        def slab_src(s):
            off = pl.multiple_of((sbase + s) * _SLAB, _SLAB)
            return table_t.at[:, pl.ds(off, _SLAB)]

        @pl.when(nsl > 0)
        def _():
            pltpu.async_copy(slab_src(0), slabs.at[0], sem_s0)

        @pl.when(nsl > 1)
        def _():
            pltpu.async_copy(slab_src(1), slabs.at[1], sem_s1)

        pltpu.sync_copy(idx_hbm, idx_all)
        iota = _iota16()

        def filt(g, cnt):
            v = idx_all[pl.ds(g * _L, _L)]
            m = (v >= lo) & (v < hi)
            packed = ((v - lo) << _POS_BITS) | (jnp.full((_L,), g * _L, jnp.int32) + iota)
            r = plsc.cumsum(m.astype(jnp.int32)) + cnt
            plsc.store_scatter(hits, [r - 1], packed, mask=m)
            return cnt + _popcnt(m)

        cnt = lax.fori_loop(0, batch // _L, filt, jnp.zeros((_L,), jnp.int32))
        ht = cnt[0]
        gg = (ht // _L) * _L
        gg = pl.multiple_of(gg, _L)
        vg = hits[pl.ds(gg, _L)]
        hits[pl.ds(gg, _L)] = jnp.where(
            iota < ht - gg, vg, jnp.full((_L,), _HUGE, jnp.int32)
        )
        ngrp = (ht + _L - 1) // _L

        def slab_count(wlo, whi):
            def body(g, c):
                v = hits[pl.ds(g * _L, _L)]
                m = (v >= wlo) & (v < whi)
                return c + _popcnt(m)

            return lax.fori_loop(0, ngrp, body, jnp.zeros((_L,), jnp.int32))[0]

        def extract_chunk(wlo, whi, c):
            rlo = c * _CHM

            def body(g, r):
                v = hits[pl.ds(g * _L, _L)]
                m = (v >= wlo) & (v < whi)
                rk = plsc.cumsum(m.astype(jnp.int32)) + r
                sel = m & (rk > rlo) & (rk <= rlo + _CHM)
                plsc.store_scatter(chunkbuf, [rk - 1 - rlo], v, mask=sel)
                return r + _popcnt(m)

            lax.fori_loop(0, ngrp, body, jnp.zeros((_L,), jnp.int32))

        def extract_count(wlo, whi):
            def body(g, r):
                v = hits[pl.ds(g * _L, _L)]
                m = (v >= wlo) & (v < whi)
                rk = plsc.cumsum(m.astype(jnp.int32)) + r
                sel = m & (rk <= _CHM)
                plsc.store_scatter(chunkbuf, [rk - 1], v, mask=sel)
                return r + _popcnt(m)

            return lax.fori_loop(0, ngrp, body, jnp.zeros((_L,), jnp.int32))[0]

        def drain_out(q_is0, fired):
            @pl.when(q_is0 & (fired == 1))
            def _():
                pltpu.make_async_copy(
                    out_hbm.at[pl.ds(0, _CHM)], staged.at[0], sem_o0
                ).wait()

        def drain_out1(q_is1, fired):
            @pl.when(q_is1 & (fired == 1))
            def _():
                pltpu.make_async_copy(
                    out_hbm.at[pl.ds(0, _CHM)], staged.at[1], sem_o1
                ).wait()

        def gather_chunk(colbase, hc, q, from_tail, p):
            lastv = plsc.load_gather(chunkbuf, [jnp.full((_L,), hc - 1, jnp.int32)])
            qv = jnp.full((_L,), q, jnp.int32)
            for g2 in range(_CHM // _L):
                lanes = jnp.full((_L,), g2 * _L, jnp.int32) + iota
                pv = chunkbuf[pl.ds(g2 * _L, _L)]
                pvf = jnp.where(lanes < hc, pv, lastv)
                col = (pvf >> _POS_BITS) - colbase
                pos = pvf & jnp.int32((1 << _POS_BITS) - 1)
                plsc.store_scatter(posbuf, [qv, jnp.zeros((_L,), jnp.int32), lanes], pos)
                pvec = jnp.full((_L,), p, jnp.int32)
                for j in range(dim):
                    jv = jnp.full((_L,), j, jnp.int32)
                    if from_tail:
                        vals = plsc.load_gather(tailslab, [jv, col])
                    else:
                        vals = plsc.load_gather(slabs, [pvec, jv, col])
                    plsc.store_scatter(staged, [qv, lanes, jv], vals)

        def fire_out(q):
            @pl.when(q == 0)
            def _():
                pltpu.async_copy(staged.at[0], out_hbm.at[posbuf.at[0, 0]], sem_o0)

            @pl.when(q == 1)
            def _():
                pltpu.async_copy(staged.at[1], out_hbm.at[posbuf.at[1, 0]], sem_o1)

        def slab_body(i, carry):
            cpar, f0, f1 = carry

            @pl.when(i % 2 == 0)
            def _():
                pltpu.make_async_copy(slab_src(i), slabs.at[0], sem_s0).wait()

            @pl.when(i % 2 == 1)
            def _():
                pltpu.make_async_copy(slab_src(i), slabs.at[1], sem_s1).wait()

            wlo = i << 23
            whi = (i + 1) << 23
            hs = extract_count(wlo, whi)
            nch = (hs + _CHM - 1) // _CHM

            def chunk_body(c, carry2):
                cpar2, g0, g1 = carry2
                q = cpar2 % 2

                @pl.when(c > 0)
                def _():
                    extract_chunk(wlo, whi, c)

                hc = jnp.minimum(hs - c * _CHM, _CHM)
                drain_out(q == 0, g0)
                drain_out1(q == 1, g1)
                gather_chunk(i * _SLAB, hc, q, False, i % 2)
                fire_out(q)
                g0n = lax.select(q == 0, jnp.int32(1), g0)
                g1n = lax.select(q == 1, jnp.int32(1), g1)
                return (cpar2 + 1, g0n, g1n)

            out_carry = lax.fori_loop(0, nch, chunk_body, (cpar, f0, f1))
            nxt = i + 2

            @pl.when((nxt < nsl) & (nxt % 2 == 0))
            def _():
                pltpu.async_copy(slab_src(nxt), slabs.at[0], sem_s0)

            @pl.when((nxt < nsl) & (nxt % 2 == 1))
            def _():
                pltpu.async_copy(slab_src(nxt), slabs.at[1], sem_s1)

            return out_carry

        cpar, f0, f1 = lax.fori_loop(
            0, nsl, slab_body, (jnp.int32(0), jnp.int32(0), jnp.int32(0))
        )

        drain_out(jnp.bool_(True), f0)
        drain_out1(jnp.bool_(True), f1)

        if tail > 0:

            @pl.when(last)
            def _():
                pltpu.sync_copy(tail_hbm, tailslab)
                wlo = (per + rem) << 23
                hs = extract_count(wlo, _HUGE)
                nch = (hs + _CHM - 1) // _CHM

                def tail_chunk(c, _):
                    @pl.when(c > 0)
                    def _():
                        extract_chunk(wlo, _HUGE, c)
                    hc = jnp.minimum(hs - c * _CHM, _CHM)
                    gather_chunk((per + rem) * _SLAB, hc, jnp.int32(0), True, 0)
                    pltpu.async_copy(
                        staged.at[0], out_hbm.at[posbuf.at[0, 0]], sem_o0
                    ).wait()
                    return 0

                lax.fori_loop(0, nch, tail_chunk, 0)

    return sweep_kernel


def kernel(categorical, table):
    (batch,) = categorical.shape
    num_classes, dim = table.shape
    fn = _make_sweep(batch, num_classes, dim)
    tcols = num_classes // 128
    tail = num_classes - tcols * 128
    tail_rows = jnp.pad(table[tcols * 128 :, :], ((0, 128 - tail), (0, 0)))
    wide = fn(
        jnp.transpose(table),
        categorical.astype(jnp.int32),
        jnp.transpose(tail_rows),
    )
    return wide[:, :dim]
